# Initial kernel scaffold; baseline (speedup 1.0000x reference)
import jax
import jax.numpy as jnp
from jax.experimental import pallas as pl


def kernel(x, edge_index, W_proj, W_skip, a_src, a_trg):
    raise NotImplementedError("write your pallas kernel here")


# speedup vs baseline: 38.7514x; 38.7514x over previous
import functools

import jax
import jax.numpy as jnp
from jax import lax
from jax.experimental import pallas as pl
from jax.experimental.pallas import tpu as pltpu
from jax.experimental.pallas import tpu_sc as plsc

N = 10000
E = 320000
FIN = 128
NH = 8
FOUT = 16
NHF = NH * FOUT
HF = NHF // 2
NHC = NH // 2

NC = 2
NS = 16
L = 16
EPT = E // NS
CH = 400
NCHUNK = EPT // CH
NP = 10240
RPT = NP // NS


def _tc1_body(x_ref, wpt_ref, wst_ref, ma_ref, mb_ref,
              proj_ref, ta_ref, tb_ref, skip_ref, s_ref):
    x = x_ref[...]
    proj = jnp.dot(x, wpt_ref[...], preferred_element_type=jnp.float32)
    proj_ref[0] = proj[:, 0:HF]
    proj_ref[1] = proj[:, HF:NHF]
    ta = jnp.dot(proj, ma_ref[...], preferred_element_type=jnp.float32)
    tb = jnp.dot(proj, mb_ref[...], preferred_element_type=jnp.float32)
    ta_ref[...] = ta
    tb_ref[...] = tb
    skip_ref[...] = jnp.dot(x, wst_ref[...], preferred_element_type=jnp.float32)
    mv = jnp.max(ta) + jnp.max(tb)
    s = jnp.where(mv >= 0, mv, 0.2 * mv)
    s_ref[...] = jnp.full((8, 128), s, jnp.float32)


_tc1 = pl.pallas_call(
    _tc1_body,
    out_shape=[
        jax.ShapeDtypeStruct((NC, N, HF), jnp.float32),
        jax.ShapeDtypeStruct((N, L), jnp.float32),
        jax.ShapeDtypeStruct((N, L), jnp.float32),
        jax.ShapeDtypeStruct((N, NHF), jnp.float32),
        jax.ShapeDtypeStruct((8, 128), jnp.float32),
    ],
)


def _sc_body(src_hbm, trg_hbm, ta_hbm, tb_hbm, proj_hbm, svec_hbm, z64_hbm,
             z16_hbm, acc_out, den_out,
             src_v, trg_v, srcp_v, ra_v, rb_v, pr_v, w_v, wf_v, s_v,
             acc_sh, den_sh, sem_g, sem_p):
    cid = lax.axis_index("c")
    sid = lax.axis_index("s")

    r0 = sid * RPT
    pltpu.sync_copy(z64_hbm.at[pl.ds(r0, RPT)], acc_sh.at[pl.ds(r0, RPT)])
    pltpu.sync_copy(z16_hbm.at[pl.ds(r0, RPT)], den_sh.at[pl.ds(r0, RPT)])
    pltpu.sync_copy(svec_hbm, s_v)
    plsc.subcore_barrier()

    s = s_v[...]
    cn = jnp.full((L,), cid * N, jnp.int32)
    hl0 = cid * NHC
    head_idx = [jnp.full((L,), h, jnp.int32) for h in range(NH)]

    def chunk_body(k, carry):
        base = sid * EPT + k * CH
        pltpu.sync_copy(src_hbm.at[pl.ds(base, CH)], src_v)
        pltpu.sync_copy(trg_hbm.at[pl.ds(base, CH)], trg_v)

        def shift_body(j, carry2):
            sl = pl.ds(j * L, L)
            srcp_v[sl] = src_v[sl] + cn
            return carry2

        lax.fori_loop(0, CH // L, shift_body, 0, unroll=4)

        ca = pltpu.async_copy(ta_hbm.at[src_v], ra_v, sem_g)
        cb = pltpu.async_copy(tb_hbm.at[trg_v], rb_v, sem_g)
        cp = pltpu.async_copy(proj_hbm.at[srcp_v], pr_v, sem_p)
        ca.wait()
        cb.wait()
        cp.wait()

        def edge_body(i, carry2):
            v = ra_v[i] + rb_v[i]
            v = jnp.where(v >= 0, v, 0.2 * v)
            w = jnp.exp(v - s)
            w_v[i] = w
            wf_v[pl.ds(i * L, L)] = w
            base_i = jnp.full((L,), i * L, jnp.int32)
            for h in range(NHC):
                wb = plsc.load_gather(wf_v, [base_i + head_idx[h] + hl0])
                sl = pl.ds(h * L, L)
                pr_v[i, sl] = pr_v[i, sl] * wb
            return carry2

        lax.fori_loop(0, CH, edge_body, 0, unroll=2)

        @pl.when(cid == 0)
        def _():
            pltpu.sync_copy(w_v, den_sh.at[trg_v], add=True)

        pltpu.sync_copy(pr_v, acc_sh.at[trg_v], add=True)
        return carry

    lax.fori_loop(0, NCHUNK, chunk_body, 0)

    plsc.subcore_barrier()
    pltpu.sync_copy(acc_sh.at[pl.ds(r0, RPT)],
                    acc_out.at[cid, pl.ds(r0, RPT)])

    @pl.when(cid == 0)
    def _():
        pltpu.sync_copy(den_sh.at[pl.ds(r0, RPT)], den_out.at[pl.ds(r0, RPT)])


_sc = functools.partial(
    pl.kernel,
    out_type=[
        jax.ShapeDtypeStruct((NC, NP, HF), jnp.float32),
        jax.ShapeDtypeStruct((NP, L), jnp.float32),
    ],
    mesh=plsc.VectorSubcoreMesh(
        core_axis_name="c", subcore_axis_name="s",
        num_cores=NC, num_subcores=NS),
    compiler_params=pltpu.CompilerParams(
        needs_layout_passes=False, use_tc_tiling_on_sc=False),
    scratch_types=[
        pltpu.VMEM((CH,), jnp.int32),
        pltpu.VMEM((CH,), jnp.int32),
        pltpu.VMEM((CH,), jnp.int32),
        pltpu.VMEM((CH, L), jnp.float32),
        pltpu.VMEM((CH, L), jnp.float32),
        pltpu.VMEM((CH, HF), jnp.float32),
        pltpu.VMEM((CH, L), jnp.float32),
        pltpu.VMEM((CH * L,), jnp.float32),
        pltpu.VMEM((L,), jnp.float32),
        pltpu.VMEM_SHARED((NP, HF), jnp.float32),
        pltpu.VMEM_SHARED((NP, L), jnp.float32),
        pltpu.SemaphoreType.DMA,
        pltpu.SemaphoreType.DMA,
    ],
)(_sc_body)


def _tc2_body(acc_ref, den_ref, skip_ref, ex_ref, out_ref):
    acc = jnp.concatenate([acc_ref[0, 0:N], acc_ref[1, 0:N]], axis=1)
    den = den_ref[0:N, 0:NH]
    den128 = jnp.dot(den, ex_ref[...], preferred_element_type=jnp.float32)
    out = acc / (den128 + 1e-16) + skip_ref[...]
    out_ref[...] = jnp.where(out > 0, out, jnp.exp(jnp.minimum(out, 0.0)) - 1.0)


_tc2 = pl.pallas_call(
    _tc2_body,
    out_shape=jax.ShapeDtypeStruct((N, NHF), jnp.float32),
)


def kernel(x, edge_index, W_proj, W_skip, a_src, a_trg):
    hh = jnp.arange(NHF, dtype=jnp.int32) // FOUT
    dup = (jnp.arange(L, dtype=jnp.int32)[None, :] % NH) == hh[:, None]
    ma = a_src.reshape(NHF)[:, None] * dup.astype(jnp.float32)
    mb = a_trg.reshape(NHF)[:, None] * dup.astype(jnp.float32)
    ex = (jnp.arange(NHF, dtype=jnp.int32)[None, :] // FOUT
          == jnp.arange(NH, dtype=jnp.int32)[:, None]).astype(jnp.float32)

    proj2, ta, tb, skip, s_splat = _tc1(x, W_proj.T, W_skip.T, ma, mb)
    svec = s_splat[0, 0:L]

    src = edge_index[0]
    trg = edge_index[1]
    proj_flat = proj2.reshape(NC * N, HF)
    z64 = jnp.zeros((NP, HF), jnp.float32)
    z16 = jnp.zeros((NP, L), jnp.float32)
    acc, den = _sc(src, trg, ta, tb, proj_flat, svec, z64, z16)

    out = _tc2(acc, den, skip, ex)
    return (out, edge_index)


# speedup vs baseline: 38.7646x; 1.0003x over previous
import functools

import jax
import jax.numpy as jnp
from jax import lax
from jax.experimental import pallas as pl
from jax.experimental.pallas import tpu as pltpu
from jax.experimental.pallas import tpu_sc as plsc

N = 10000
E = 320000
FIN = 128
NH = 8
FOUT = 16
NHF = NH * FOUT
HF = NHF // 2
NHC = NH // 2

NC = 2
NS = 16
L = 16
EPT = E // NS
CH = 200
NCHUNK = EPT // CH
NP = 10240
RPT = NP // NS


def _tc1_body(x_ref, wpt_ref, wst_ref, ma_ref, mb_ref,
              proj_ref, ta_ref, tb_ref, skip_ref, s_ref):
    x = x_ref[...]
    proj = jnp.dot(x, wpt_ref[...], preferred_element_type=jnp.float32)
    proj_ref[0] = proj[:, 0:HF]
    proj_ref[1] = proj[:, HF:NHF]
    ta = jnp.dot(proj, ma_ref[...], preferred_element_type=jnp.float32)
    tb = jnp.dot(proj, mb_ref[...], preferred_element_type=jnp.float32)
    ta_ref[...] = ta
    tb_ref[...] = tb
    skip_ref[...] = jnp.dot(x, wst_ref[...], preferred_element_type=jnp.float32)
    mv = jnp.max(ta) + jnp.max(tb)
    s = jnp.where(mv >= 0, mv, 0.2 * mv)
    s_ref[...] = jnp.full((8, 128), s, jnp.float32)


_tc1 = pl.pallas_call(
    _tc1_body,
    out_shape=[
        jax.ShapeDtypeStruct((NC, N, HF), jnp.float32),
        jax.ShapeDtypeStruct((N, L), jnp.float32),
        jax.ShapeDtypeStruct((N, L), jnp.float32),
        jax.ShapeDtypeStruct((N, NHF), jnp.float32),
        jax.ShapeDtypeStruct((8, 128), jnp.float32),
    ],
)


def _sc_body(src_hbm, trg_hbm, srcp_hbm, ta_hbm, tb_hbm, proj_hbm, svec_hbm,
             z64_hbm, z16_hbm, acc_out, den_out,
             src_v, trg_v, srcp_v, ra_v, rb_v, pr_v, w_v, wf_v, s_v,
             acc_sh, den_sh, sem_g0, sem_g1, sem_p0, sem_p1):
    cid = lax.axis_index("c")
    sid = lax.axis_index("s")

    r0 = sid * RPT
    pltpu.sync_copy(z64_hbm.at[pl.ds(r0, RPT)], acc_sh.at[pl.ds(r0, RPT)])
    pltpu.sync_copy(z16_hbm.at[pl.ds(r0, RPT)], den_sh.at[pl.ds(r0, RPT)])
    pltpu.sync_copy(svec_hbm, s_v)
    plsc.subcore_barrier()

    s = s_v[...]
    hl0 = cid * NHC
    head_idx = [jnp.full((L,), h, jnp.int32) for h in range(NH)]

    def sems(b):
        return (sem_g0, sem_p0) if b == 0 else (sem_g1, sem_p1)

    def prefetch(k, b):
        sg, sp = sems(b)
        base = sid * EPT + k * CH
        pltpu.sync_copy(src_hbm.at[pl.ds(base, CH)], src_v.at[b])
        pltpu.sync_copy(trg_hbm.at[pl.ds(base, CH)], trg_v.at[b])
        pltpu.sync_copy(srcp_hbm.at[pl.ds(cid * E + base, CH)], srcp_v.at[b])
        pltpu.async_copy(ta_hbm.at[src_v.at[b]], ra_v.at[b], sg)
        pltpu.async_copy(tb_hbm.at[trg_v.at[b]], rb_v.at[b], sg)
        pltpu.async_copy(proj_hbm.at[srcp_v.at[b]], pr_v.at[b], sp)

    def process(k, b):
        sg, sp = sems(b)
        pltpu.make_async_copy(ta_hbm.at[src_v.at[b]], ra_v.at[b], sg).wait()
        pltpu.make_async_copy(tb_hbm.at[trg_v.at[b]], rb_v.at[b], sg).wait()
        pltpu.make_async_copy(proj_hbm.at[srcp_v.at[b]], pr_v.at[b], sp).wait()

        @pl.when(k + 1 < NCHUNK)
        def _():
            prefetch(k + 1, 1 - b)

        def edge_body(i, carry2):
            v = ra_v[b, i] + rb_v[b, i]
            v = jnp.where(v >= 0, v, 0.2 * v)
            w = jnp.exp(v - s)
            w_v[i] = w
            wf_v[pl.ds(i * L, L)] = w
            base_i = jnp.full((L,), i * L, jnp.int32)
            for h in range(NHC):
                wb = plsc.load_gather(wf_v, [base_i + head_idx[h] + hl0])
                sl = pl.ds(h * L, L)
                pr_v[b, i, sl] = pr_v[b, i, sl] * wb
            return carry2

        lax.fori_loop(0, CH, edge_body, 0, unroll=2)

        @pl.when(cid == 0)
        def _():
            pltpu.sync_copy(w_v, den_sh.at[trg_v.at[b]], add=True)

        pltpu.sync_copy(pr_v.at[b], acc_sh.at[trg_v.at[b]], add=True)

    prefetch(0, 0)

    def pair_body(p, carry):
        process(2 * p, 0)
        process(2 * p + 1, 1)
        return carry

    lax.fori_loop(0, NCHUNK // 2, pair_body, 0)

    plsc.subcore_barrier()
    pltpu.sync_copy(acc_sh.at[pl.ds(r0, RPT)],
                    acc_out.at[cid, pl.ds(r0, RPT)])

    @pl.when(cid == 0)
    def _():
        pltpu.sync_copy(den_sh.at[pl.ds(r0, RPT)], den_out.at[pl.ds(r0, RPT)])


_sc = functools.partial(
    pl.kernel,
    out_type=[
        jax.ShapeDtypeStruct((NC, NP, HF), jnp.float32),
        jax.ShapeDtypeStruct((NP, L), jnp.float32),
    ],
    mesh=plsc.VectorSubcoreMesh(
        core_axis_name="c", subcore_axis_name="s",
        num_cores=NC, num_subcores=NS),
    compiler_params=pltpu.CompilerParams(
        needs_layout_passes=False, use_tc_tiling_on_sc=False),
    scratch_types=[
        pltpu.VMEM((2, CH), jnp.int32),
        pltpu.VMEM((2, CH), jnp.int32),
        pltpu.VMEM((2, CH), jnp.int32),
        pltpu.VMEM((2, CH, L), jnp.float32),
        pltpu.VMEM((2, CH, L), jnp.float32),
        pltpu.VMEM((2, CH, HF), jnp.float32),
        pltpu.VMEM((CH, L), jnp.float32),
        pltpu.VMEM((CH * L,), jnp.float32),
        pltpu.VMEM((L,), jnp.float32),
        pltpu.VMEM_SHARED((NP, HF), jnp.float32),
        pltpu.VMEM_SHARED((NP, L), jnp.float32),
        pltpu.SemaphoreType.DMA,
        pltpu.SemaphoreType.DMA,
        pltpu.SemaphoreType.DMA,
        pltpu.SemaphoreType.DMA,
    ],
)(_sc_body)


def _tc2_body(acc_ref, den_ref, skip_ref, ex_ref, out_ref):
    acc = jnp.concatenate([acc_ref[0, 0:N], acc_ref[1, 0:N]], axis=1)
    den = den_ref[0:N, 0:NH]
    den128 = jnp.dot(den, ex_ref[...], preferred_element_type=jnp.float32)
    out = acc / (den128 + 1e-16) + skip_ref[...]
    out_ref[...] = jnp.where(out > 0, out, jnp.exp(jnp.minimum(out, 0.0)) - 1.0)


_tc2 = pl.pallas_call(
    _tc2_body,
    out_shape=jax.ShapeDtypeStruct((N, NHF), jnp.float32),
)


def kernel(x, edge_index, W_proj, W_skip, a_src, a_trg):
    hh = jnp.arange(NHF, dtype=jnp.int32) // FOUT
    dup = (jnp.arange(L, dtype=jnp.int32)[None, :] % NH) == hh[:, None]
    ma = a_src.reshape(NHF)[:, None] * dup.astype(jnp.float32)
    mb = a_trg.reshape(NHF)[:, None] * dup.astype(jnp.float32)
    ex = (jnp.arange(NHF, dtype=jnp.int32)[None, :] // FOUT
          == jnp.arange(NH, dtype=jnp.int32)[:, None]).astype(jnp.float32)

    proj2, ta, tb, skip, s_splat = _tc1(x, W_proj.T, W_skip.T, ma, mb)
    svec = s_splat[0, 0:L]

    src = edge_index[0]
    trg = edge_index[1]
    srcp = jnp.concatenate([src, src + N])
    proj_flat = proj2.reshape(NC * N, HF)
    z64 = jnp.zeros((NP, HF), jnp.float32)
    z16 = jnp.zeros((NP, L), jnp.float32)
    acc, den = _sc(src, trg, srcp, ta, tb, proj_flat, svec, z64, z16)

    out = _tc2(acc, den, skip, ex)
    return (out, edge_index)


# speedup vs baseline: 89.9214x; 2.3197x over previous
import functools

import jax
import jax.numpy as jnp
from jax import lax
from jax.experimental import pallas as pl
from jax.experimental.pallas import tpu as pltpu
from jax.experimental.pallas import tpu_sc as plsc

N = 10000
E = 320000
FIN = 128
NH = 8
FOUT = 16
NHF = NH * FOUT
HF = NHF // 2
NHC = NH // 2

NC = 2
NS = 16
L = 16
EPT = E // NS
CH = 200
NCHUNK = EPT // CH
NP = 10240
RPT = NP // NS


def _tc1_body(x_ref, wpt_ref, wst_ref, ma_ref, mb_ref,
              proj_ref, ta_ref, tb_ref, skip_ref, s_ref):
    x = x_ref[...]
    proj = jnp.dot(x, wpt_ref[...], preferred_element_type=jnp.float32)
    proj_ref[0] = proj[:, 0:HF]
    proj_ref[1] = proj[:, HF:NHF]
    ta = jnp.dot(proj, ma_ref[...], preferred_element_type=jnp.float32)
    tb = jnp.dot(proj, mb_ref[...], preferred_element_type=jnp.float32)
    ta_ref[...] = ta
    tb_ref[...] = tb
    skip_ref[...] = jnp.dot(x, wst_ref[...], preferred_element_type=jnp.float32)
    mv = jnp.max(ta) + jnp.max(tb)
    s = jnp.where(mv >= 0, mv, 0.2 * mv)
    s_ref[...] = jnp.full((8, 128), s, jnp.float32)


_tc1 = pl.pallas_call(
    _tc1_body,
    out_shape=[
        jax.ShapeDtypeStruct((NC, N, HF), jnp.float32),
        jax.ShapeDtypeStruct((N, L), jnp.float32),
        jax.ShapeDtypeStruct((N, L), jnp.float32),
        jax.ShapeDtypeStruct((N, NHF), jnp.float32),
        jax.ShapeDtypeStruct((8, 128), jnp.float32),
    ],
)


def _sc_body(src_hbm, trg_hbm, srcp_hbm, ta_hbm, tb_hbm, proj_hbm, svec_hbm,
             z64_hbm, z16_hbm, acc_out, den_out,
             src_v, trg_v, srcp_v, ra_v, rb_v, pr_v, w_v, wf_v, s_v,
             acc_sh, den_sh, sem_g0, sem_g1, sem_p0, sem_p1,
             sem_a0, sem_a1, sem_d0, sem_d1):
    cid = lax.axis_index("c")
    sid = lax.axis_index("s")

    r0 = sid * RPT
    pltpu.sync_copy(z64_hbm.at[pl.ds(r0, RPT)], acc_sh.at[pl.ds(r0, RPT)])
    pltpu.sync_copy(z16_hbm.at[pl.ds(r0, RPT)], den_sh.at[pl.ds(r0, RPT)])
    pltpu.sync_copy(svec_hbm, s_v)
    plsc.subcore_barrier()

    s = s_v[...]
    hl0 = cid * NHC
    head_idx = [jnp.full((L,), h, jnp.int32) for h in range(NH)]

    def sems(b):
        if b == 0:
            return (sem_g0, sem_p0, sem_a0, sem_d0)
        return (sem_g1, sem_p1, sem_a1, sem_d1)

    def prefetch(k, b):
        sg, sp, sa, sd = sems(b)

        @pl.when(k >= 2)
        def _():
            pltpu.make_async_copy(pr_v.at[b], acc_sh.at[trg_v.at[b]],
                                  sa).wait()

            @pl.when(cid == 0)
            def _():
                pltpu.make_async_copy(w_v.at[b], den_sh.at[trg_v.at[b]],
                                      sd).wait()

        base = sid * EPT + k * CH
        pltpu.sync_copy(src_hbm.at[pl.ds(base, CH)], src_v.at[b])
        pltpu.sync_copy(trg_hbm.at[pl.ds(base, CH)], trg_v.at[b])
        pltpu.sync_copy(srcp_hbm.at[pl.ds(cid * E + base, CH)], srcp_v.at[b])
        pltpu.async_copy(ta_hbm.at[src_v.at[b]], ra_v.at[b], sg)
        pltpu.async_copy(tb_hbm.at[trg_v.at[b]], rb_v.at[b], sg)
        pltpu.async_copy(proj_hbm.at[srcp_v.at[b]], pr_v.at[b], sp)

    def process(k, b):
        sg, sp, sa, sd = sems(b)
        pltpu.make_async_copy(ta_hbm.at[src_v.at[b]], ra_v.at[b], sg).wait()
        pltpu.make_async_copy(tb_hbm.at[trg_v.at[b]], rb_v.at[b], sg).wait()
        pltpu.make_async_copy(proj_hbm.at[srcp_v.at[b]], pr_v.at[b], sp).wait()

        @pl.when(k + 1 < NCHUNK)
        def _():
            prefetch(k + 1, 1 - b)

        @plsc.parallel_loop(0, CH, unroll=8)
        def _(i):
            v = ra_v[b, i] + rb_v[b, i]
            v = jnp.where(v >= 0, v, 0.2 * v)
            w = jnp.exp(v - s)
            w_v[b, i] = w
            wf_v[b, pl.ds(i * L, L)] = w

        @plsc.parallel_loop(0, CH, unroll=8)
        def _(i):
            base_i = jnp.full((L,), i * L, jnp.int32)
            for h in range(NHC):
                wb = plsc.load_gather(wf_v.at[b],
                                      [base_i + head_idx[h] + hl0])
                sl = pl.ds(h * L, L)
                pr_v[b, i, sl] = pr_v[b, i, sl] * wb

        @pl.when(cid == 0)
        def _():
            pltpu.async_copy(w_v.at[b], den_sh.at[trg_v.at[b]], sd, add=True)

        pltpu.async_copy(pr_v.at[b], acc_sh.at[trg_v.at[b]], sa, add=True)

    prefetch(0, 0)

    def pair_body(p, carry):
        process(2 * p, 0)
        process(2 * p + 1, 1)
        return carry

    lax.fori_loop(0, NCHUNK // 2, pair_body, 0)

    for b in (0, 1):
        sg, sp, sa, sd = sems(b)
        pltpu.make_async_copy(pr_v.at[b], acc_sh.at[trg_v.at[b]], sa).wait()

        @pl.when(cid == 0)
        def _():
            pltpu.make_async_copy(w_v.at[b], den_sh.at[trg_v.at[b]],
                                  sd).wait()

    plsc.subcore_barrier()
    pltpu.sync_copy(acc_sh.at[pl.ds(r0, RPT)],
                    acc_out.at[cid, pl.ds(r0, RPT)])

    @pl.when(cid == 0)
    def _():
        pltpu.sync_copy(den_sh.at[pl.ds(r0, RPT)], den_out.at[pl.ds(r0, RPT)])


_sc = functools.partial(
    pl.kernel,
    out_type=[
        jax.ShapeDtypeStruct((NC, NP, HF), jnp.float32),
        jax.ShapeDtypeStruct((NP, L), jnp.float32),
    ],
    mesh=plsc.VectorSubcoreMesh(
        core_axis_name="c", subcore_axis_name="s",
        num_cores=NC, num_subcores=NS),
    compiler_params=pltpu.CompilerParams(
        needs_layout_passes=False, use_tc_tiling_on_sc=False),
    scratch_types=[
        pltpu.VMEM((2, CH), jnp.int32),
        pltpu.VMEM((2, CH), jnp.int32),
        pltpu.VMEM((2, CH), jnp.int32),
        pltpu.VMEM((2, CH, L), jnp.float32),
        pltpu.VMEM((2, CH, L), jnp.float32),
        pltpu.VMEM((2, CH, HF), jnp.float32),
        pltpu.VMEM((2, CH, L), jnp.float32),
        pltpu.VMEM((2, CH * L), jnp.float32),
        pltpu.VMEM((L,), jnp.float32),
        pltpu.VMEM_SHARED((NP, HF), jnp.float32),
        pltpu.VMEM_SHARED((NP, L), jnp.float32),
        pltpu.SemaphoreType.DMA,
        pltpu.SemaphoreType.DMA,
        pltpu.SemaphoreType.DMA,
        pltpu.SemaphoreType.DMA,
        pltpu.SemaphoreType.DMA,
        pltpu.SemaphoreType.DMA,
        pltpu.SemaphoreType.DMA,
        pltpu.SemaphoreType.DMA,
    ],
)(_sc_body)


def _tc2_body(acc_ref, den_ref, skip_ref, ex_ref, out_ref):
    acc = jnp.concatenate([acc_ref[0, 0:N], acc_ref[1, 0:N]], axis=1)
    den = den_ref[0:N, 0:NH]
    den128 = jnp.dot(den, ex_ref[...], preferred_element_type=jnp.float32)
    out = acc / (den128 + 1e-16) + skip_ref[...]
    out_ref[...] = jnp.where(out > 0, out, jnp.exp(jnp.minimum(out, 0.0)) - 1.0)


_tc2 = pl.pallas_call(
    _tc2_body,
    out_shape=jax.ShapeDtypeStruct((N, NHF), jnp.float32),
)


def kernel(x, edge_index, W_proj, W_skip, a_src, a_trg):
    hh = jnp.arange(NHF, dtype=jnp.int32) // FOUT
    dup = (jnp.arange(L, dtype=jnp.int32)[None, :] % NH) == hh[:, None]
    ma = a_src.reshape(NHF)[:, None] * dup.astype(jnp.float32)
    mb = a_trg.reshape(NHF)[:, None] * dup.astype(jnp.float32)
    ex = (jnp.arange(NHF, dtype=jnp.int32)[None, :] // FOUT
          == jnp.arange(NH, dtype=jnp.int32)[:, None]).astype(jnp.float32)

    proj2, ta, tb, skip, s_splat = _tc1(x, W_proj.T, W_skip.T, ma, mb)
    svec = s_splat[0, 0:L]

    src = edge_index[0]
    trg = edge_index[1]
    srcp = jnp.concatenate([src, src + N])
    proj_flat = proj2.reshape(NC * N, HF)
    z64 = jnp.zeros((NP, HF), jnp.float32)
    z16 = jnp.zeros((NP, L), jnp.float32)
    acc, den = _sc(src, trg, srcp, ta, tb, proj_flat, svec, z64, z16)

    out = _tc2(acc, den, skip, ex)
    return (out, edge_index)
